# baseline (speedup 1.0000x reference)
import dataclasses
import functools

import jax
import jax.numpy as jnp
from jax import lax
from jax.experimental import pallas as pl
from jax.experimental.pallas import tpu as pltpu
from jax.experimental.pallas import tpu_sc as plsc

_NC = 2
_NS = 16
_L = 16
_NW = _NC * _NS

_NEG = -(2**31) + 1


def _lane_extract(vec, lane, iota):
    return jnp.max(jnp.where(iota == lane, vec, jnp.int32(_NEG)))


@functools.cache
def _make_sc_kernel(n_rows, d, b):
    rows_per_w = (n_rows // (_NW * 8)) * 8
    last_rows = n_rows - rows_per_w * (_NW - 1)
    tbl = ((last_rows + _L - 1) // _L) * _L + _L

    mesh = plsc.VectorSubcoreMesh(core_axis_name="c", subcore_axis_name="s")
    cp = dataclasses.replace(
        pltpu.CompilerParams(), needs_layout_passes=False)

    @functools.partial(
        pl.kernel,
        out_type=jax.ShapeDtypeStruct((n_rows, d), jnp.float32),
        mesh=mesh,
        compiler_params=cp,
        scratch_types=[
            pltpu.VMEM((b,), jnp.int32),
            pltpu.VMEM((tbl,), jnp.int32),
            pltpu.VMEM((tbl,), jnp.int32),
            pltpu.VMEM((tbl,), jnp.int32),
            pltpu.SemaphoreType.DMA,
            pltpu.SemaphoreType.DMA,
        ],
    )
    def sc_kernel(bank_hbm, val_hbm, idx_hbm, out_hbm,
                  idx_v, table, rlist, wlist, sem_c, sem_r):
        iota = lax.iota(jnp.int32, _L)
        wid = lax.axis_index("s") * _NC + lax.axis_index("c")
        a = wid * rows_per_w
        n = jnp.where(wid == _NW - 1, last_rows, rows_per_w)

        copy_desc = pltpu.async_copy(
            bank_hbm.at[pl.ds(a, n)], out_hbm.at[pl.ds(a, n)], sem_c)

        pltpu.sync_copy(idx_hbm, idx_v)

        @pl.loop(0, b, step=_L)
        def _(i):
            v = idx_v[pl.ds(i, _L)]
            m = (v >= a) & (v < a + n)
            plsc.store_scatter(table, [v - a], iota + i, mask=m)

        def w_cond(changed):
            return changed > 0

        def w_body(_):
            @pl.loop(0, b, step=_L, init_carry=jnp.int32(0))
            def changed(i, ch):
                v = idx_v[pl.ds(i, _L)]
                pos = iota + i
                m = (v >= a) & (v < a + n)
                t = plsc.load_gather(table, [v - a], mask=m)
                m2 = m & (pos > t)
                plsc.store_scatter(table, [v - a], pos, mask=m2)
                return ch + jnp.sum(m2.astype(jnp.int32))

            return changed

        lax.while_loop(w_cond, w_body, jnp.int32(1))

        @pl.loop(0, b, step=_L, init_carry=jnp.int32(0))
        def cnt(i, c):
            v = idx_v[pl.ds(i, _L)]
            pos = iota + i
            m = (v >= a) & (v < a + n)
            t = plsc.load_gather(table, [v - a], mask=m)
            mw = m & (pos == t)
            k = mw.astype(jnp.int32)
            off = plsc.cumsum(k) - 1 + c
            plsc.store_scatter(rlist, [off], v, mask=mw)
            plsc.store_scatter(wlist, [off], pos, mask=mw)
            return c + jnp.sum(k)

        @pl.when(cnt > 0)
        def _():
            cnt_pad = ((cnt + _L - 1) // _L) * _L
            lastv = rlist[pl.ds(((cnt - 1) // _L) * _L, _L)]
            lastw = wlist[pl.ds(((cnt - 1) // _L) * _L, _L)]
            lane = (cnt - 1) % _L
            r_last = _lane_extract(lastv, lane, iota)
            w_last = _lane_extract(lastw, lane, iota)
            mp = iota + cnt < cnt_pad
            plsc.store_scatter(rlist, [iota + cnt],
                               jnp.full((_L,), r_last, jnp.int32), mask=mp)
            plsc.store_scatter(wlist, [iota + cnt],
                               jnp.full((_L,), w_last, jnp.int32), mask=mp)

            copy_desc.wait()

            @pl.loop(0, cnt_pad, step=_L)
            def _(j):
                rv = rlist[pl.ds(j, _L)]
                wv = wlist[pl.ds(j, _L)]
                descs = []
                for lane_i in range(_L):
                    r = _lane_extract(rv, lane_i, iota)
                    w = _lane_extract(wv, lane_i, iota)
                    descs.append(pltpu.async_copy(
                        val_hbm.at[w], out_hbm.at[r], sem_r))
                for desc in descs:
                    desc.wait()

        @pl.when(cnt == 0)
        def _():
            copy_desc.wait()

    return sc_kernel


def kernel(bank, idx, val):
    n, d = bank.shape
    return _make_sc_kernel(n, d, idx.shape[0])(bank, val, idx)


# speedup vs baseline: 8.4581x; 8.4581x over previous
import dataclasses
import functools

import jax
import jax.numpy as jnp
from jax import lax
from jax.experimental import pallas as pl
from jax.experimental.pallas import tpu as pltpu
from jax.experimental.pallas import tpu_sc as plsc

_NC = 2
_NS = 16
_L = 16
_NW = _NC * _NS

_NEG = -(2**31) + 1


def _lane_extract(vec, lane, iota):
    return jnp.max(jnp.where(iota == lane, vec, jnp.int32(_NEG)))


@functools.cache
def _make_sc_kernel(n_rows, d, b):
    rows_per_w = (n_rows // (_NW * 8)) * 8
    last_rows = n_rows - rows_per_w * (_NW - 1)
    tbl = ((last_rows + _L - 1) // _L) * _L + _L
    nch = 10
    ch = rows_per_w // nch
    left = last_rows - rows_per_w
    assert rows_per_w % nch == 0 and ch % 8 == 0 and 0 <= left <= ch

    mesh = plsc.VectorSubcoreMesh(core_axis_name="c", subcore_axis_name="s")
    cp = dataclasses.replace(
        pltpu.CompilerParams(), needs_layout_passes=False)

    @functools.partial(
        pl.kernel,
        out_type=jax.ShapeDtypeStruct((n_rows, d), jnp.float32),
        mesh=mesh,
        compiler_params=cp,
        scratch_types=[
            pltpu.VMEM((b,), jnp.int32),
            pltpu.VMEM((tbl,), jnp.int32),
            pltpu.VMEM((tbl,), jnp.int32),
            pltpu.VMEM((tbl,), jnp.int32),
            pltpu.VMEM((ch, d), jnp.float32),
            pltpu.VMEM((ch, d), jnp.float32),
            pltpu.SemaphoreType.DMA,
            pltpu.SemaphoreType.DMA,
            pltpu.SemaphoreType.DMA,
        ],
    )
    def sc_kernel(bank_hbm, val_hbm, idx_hbm, out_hbm,
                  idx_v, table, rlist, wlist, buf0, buf1,
                  sem_i, sem_o, sem_r):
        iota = lax.iota(jnp.int32, _L)
        wid = lax.axis_index("s") * _NC + lax.axis_index("c")
        a = wid * rows_per_w
        n = jnp.where(wid == _NW - 1, last_rows, rows_per_w)
        bufs = (buf0, buf1)

        in_descs = {}
        for c in range(2):
            in_descs[c] = pltpu.async_copy(
                bank_hbm.at[pl.ds(a + c * ch, ch)], bufs[c % 2], sem_i)

        pltpu.sync_copy(idx_hbm, idx_v)

        @pl.loop(0, b, step=_L)
        def _(i):
            v = idx_v[pl.ds(i, _L)]
            m = (v >= a) & (v < a + n)
            plsc.store_scatter(table, [v - a], iota + i, mask=m)

        def w_cond(changed):
            return changed > 0

        def w_body(_):
            @pl.loop(0, b, step=_L, init_carry=jnp.int32(0))
            def changed(i, ch):
                v = idx_v[pl.ds(i, _L)]
                pos = iota + i
                m = (v >= a) & (v < a + n)
                t = plsc.load_gather(table, [v - a], mask=m)
                m2 = m & (pos > t)
                plsc.store_scatter(table, [v - a], pos, mask=m2)
                return ch + jnp.sum(m2.astype(jnp.int32))

            return changed

        lax.while_loop(w_cond, w_body, jnp.int32(1))

        @pl.loop(0, b, step=_L, init_carry=jnp.int32(0))
        def cnt(i, c):
            v = idx_v[pl.ds(i, _L)]
            pos = iota + i
            m = (v >= a) & (v < a + n)
            t = plsc.load_gather(table, [v - a], mask=m)
            mw = m & (pos == t)
            k = mw.astype(jnp.int32)
            off = plsc.cumsum(k) - 1 + c
            plsc.store_scatter(rlist, [off], v, mask=mw)
            plsc.store_scatter(wlist, [off], pos, mask=mw)
            return c + jnp.sum(k)

        for c in range(nch):
            in_descs[c].wait()
            out_desc = pltpu.async_copy(
                bufs[c % 2], out_hbm.at[pl.ds(a + c * ch, ch)], sem_o)
            out_desc.wait()
            if c + 2 < nch:
                in_descs[c + 2] = pltpu.async_copy(
                    bank_hbm.at[pl.ds(a + (c + 2) * ch, ch)],
                    bufs[c % 2], sem_i)
        if left > 0:
            @pl.when(wid == _NW - 1)
            def _():
                lo = a + nch * ch
                pltpu.sync_copy(bank_hbm.at[pl.ds(lo, left)],
                                buf0.at[pl.ds(0, left)])
                pltpu.sync_copy(buf0.at[pl.ds(0, left)],
                                out_hbm.at[pl.ds(lo, left)])

        @pl.when(cnt > 0)
        def _():
            cnt_pad = ((cnt + _L - 1) // _L) * _L
            lastv = rlist[pl.ds(((cnt - 1) // _L) * _L, _L)]
            lastw = wlist[pl.ds(((cnt - 1) // _L) * _L, _L)]
            lane = (cnt - 1) % _L
            r_last = _lane_extract(lastv, lane, iota)
            w_last = _lane_extract(lastw, lane, iota)
            mp = iota + cnt < cnt_pad
            plsc.store_scatter(rlist, [iota + cnt],
                               jnp.full((_L,), r_last, jnp.int32), mask=mp)
            plsc.store_scatter(wlist, [iota + cnt],
                               jnp.full((_L,), w_last, jnp.int32), mask=mp)

            @pl.loop(0, cnt_pad, step=_L)
            def _(j):
                rv = rlist[pl.ds(j, _L)]
                wv = wlist[pl.ds(j, _L)]
                descs = []
                for lane_i in range(_L):
                    r = _lane_extract(rv, lane_i, iota)
                    w = _lane_extract(wv, lane_i, iota)
                    descs.append(pltpu.async_copy(
                        val_hbm.at[w], out_hbm.at[r], sem_r))
                for desc in descs:
                    desc.wait()

    return sc_kernel


def kernel(bank, idx, val):
    n, d = bank.shape
    return _make_sc_kernel(n, d, idx.shape[0])(bank, val, idx)


# speedup vs baseline: 10.9065x; 1.2895x over previous
import dataclasses
import functools

import jax
import jax.numpy as jnp
from jax import lax
from jax.experimental import pallas as pl
from jax.experimental.pallas import tpu as pltpu
from jax.experimental.pallas import tpu_sc as plsc

_NC = 2
_NS = 16
_L = 16
_NW = _NC * _NS

_COPY_BLOCK = 10000


def _copy_body(x_ref, o_ref):
    o_ref[...] = x_ref[...]


def _tc_copy(bank):
    n, d = bank.shape
    blk = _COPY_BLOCK
    assert n % blk == 0
    return pl.pallas_call(
        _copy_body,
        grid=(n // blk,),
        in_specs=[pl.BlockSpec((blk, d), lambda i: (i, 0))],
        out_specs=pl.BlockSpec((blk, d), lambda i: (i, 0)),
        out_shape=jax.ShapeDtypeStruct((n, d), bank.dtype),
    )(bank)


@functools.cache
def _make_sc_scatter(n_rows, d, b):
    w = b // _NW
    assert b % (_NW * _L) == 0

    mesh = plsc.VectorSubcoreMesh(core_axis_name="c", subcore_axis_name="s")
    cp = pltpu.CompilerParams()
    fields = pltpu.CompilerParams.__dataclass_fields__
    if "needs_layout_passes" in fields:
        cp = dataclasses.replace(cp, needs_layout_passes=False)
    if "use_tc_tiling_on_sc" in fields:
        cp = dataclasses.replace(cp, use_tc_tiling_on_sc=False)

    @functools.partial(
        pl.kernel,
        out_type=(),
        mesh=mesh,
        compiler_params=cp,
        scratch_types=[
            pltpu.VMEM((b,), jnp.int32),
            pltpu.VMEM((n_rows,), jnp.int32),
            pltpu.VMEM((w,), jnp.int32),
            pltpu.VMEM((w,), jnp.int32),
            pltpu.VMEM((w, d), jnp.float32),
            pltpu.SemaphoreType.DMA,
        ],
    )
    def sc_scatter(val_hbm, idx_hbm, out_hbm,
                   idx_buf, pos_table, win_buf, idx_win, rows_v, sem):
        iota = lax.iota(jnp.int32, _L)
        wid = lax.axis_index("s") * _NC + lax.axis_index("c")
        base = wid * w

        pltpu.sync_copy(idx_hbm, idx_buf)

        @pl.loop(0, b, step=_L)
        def _(i):
            v = idx_buf[pl.ds(i, _L)]
            plsc.store_scatter(pos_table, [v], iota + i)

        def w_cond(changed):
            return changed > 0

        def w_body(_):
            @pl.loop(0, b, step=_L, init_carry=jnp.int32(0))
            def changed(i, ch):
                v = idx_buf[pl.ds(i, _L)]
                pos = iota + i
                t = plsc.load_gather(pos_table, [v])
                m = pos > t
                plsc.store_scatter(pos_table, [v], pos, mask=m)
                return ch + jnp.sum(m.astype(jnp.int32))

            return changed

        lax.while_loop(w_cond, w_body, jnp.int32(1))

        @pl.loop(0, w, step=_L)
        def _(r):
            v = idx_buf[pl.ds(base + r, _L)]
            win_buf[pl.ds(r, _L)] = plsc.load_gather(pos_table, [v])

        pltpu.sync_copy(idx_hbm.at[pl.ds(base, w)], idx_win)
        pltpu.async_copy(val_hbm.at[win_buf], rows_v, sem).wait()
        pltpu.async_copy(rows_v, out_hbm.at[idx_win], sem).wait()

    return sc_scatter


def kernel(bank, idx, val):
    n, d = bank.shape
    out_ref = jax.new_ref(bank)
    _make_sc_scatter(n, d, idx.shape[0])(val, idx, out_ref)
    return out_ref[...]
